# baseline (speedup 1.0000x reference)
import functools

import jax
import jax.numpy as jnp
from jax import lax
from jax.experimental import pallas as pl
from jax.experimental.pallas import tpu as pltpu
from jax.experimental.pallas import tpu_sc as plsc

_NUM_CORES = 2
_NUM_SUBCORES = 16
_NW = _NUM_CORES * _NUM_SUBCORES
_B = 2000
_UNROLL = 5


def _make_sc_call(n_nodes, n_edges):
    e_per = n_edges // _NW
    n_chunks = e_per // _B
    mesh = plsc.VectorSubcoreMesh(
        core_axis_name="c", subcore_axis_name="s",
        num_cores=_NUM_CORES, num_subcores=_NUM_SUBCORES)

    @functools.partial(
        pl.kernel,
        out_type=tuple(
            jax.ShapeDtypeStruct((n_edges,), jnp.float32) for _ in range(3)),
        mesh=mesh,
        compiler_params=pltpu.CompilerParams(needs_layout_passes=False),
        scratch_types=[
            [pltpu.VMEM((_B,), jnp.int32)] * 2,
            [pltpu.VMEM((_B,), jnp.int32)] * 2,
            [pltpu.VMEM((_B,), jnp.float32)] * 2,
            [pltpu.VMEM((_B,), jnp.float32)] * 2,
            pltpu.VMEM((n_nodes,), jnp.int32),
            [pltpu.SemaphoreType.DMA] * 2,
            [pltpu.SemaphoreType.DMA] * 2,
        ],
    )
    def sc_call(pxy, pzi, sidx, didx, dxo, dyo, dzo,
                si_v, di_v, dfa_v, dfb_v, tab_v, sem_i, sem_o):
        wid = lax.axis_index("s") * _NUM_CORES + lax.axis_index("c")

        def start_idx(g, b):
            base = wid * e_per + g * _B
            pltpu.async_copy(sidx.at[pl.ds(base, _B)], si_v[b], sem_i[b])
            pltpu.async_copy(didx.at[pl.ds(base, _B)], di_v[b], sem_i[b])

        def wait_idx(b):
            pltpu.make_async_copy(sidx.at[pl.ds(0, _B)], si_v[b],
                                  sem_i[b]).wait()
            pltpu.make_async_copy(didx.at[pl.ds(0, _B)], di_v[b],
                                  sem_i[b]).wait()

        def wait_out(b, outs):
            for o, dv in zip(outs, (dfa_v, dfb_v)):
                pltpu.make_async_copy(dv[b], o.at[pl.ds(0, _B)],
                                      sem_o[b]).wait()

        def xy_group(b, k):
            si = si_v[b][pl.ds(16 * k, 16)]
            di = di_v[b][pl.ds(16 * k, 16)]
            sp = plsc.load_gather(tab_v, [si])
            dp = plsc.load_gather(tab_v, [di])
            hi = jnp.int32(-65536)
            sx = plsc.bitcast(sp & hi, jnp.float32)
            dxv = plsc.bitcast(dp & hi, jnp.float32)
            sy = plsc.bitcast(lax.shift_left(sp, 16), jnp.float32)
            dyv = plsc.bitcast(lax.shift_left(dp, 16), jnp.float32)
            dfa_v[b][pl.ds(16 * k, 16)] = dxv - sx
            dfb_v[b][pl.ds(16 * k, 16)] = dyv - sy

        def z_group(b, k):
            si = si_v[b][pl.ds(16 * k, 16)]
            di = di_v[b][pl.ds(16 * k, 16)]
            sv = plsc.bitcast(plsc.load_gather(tab_v, [si]), jnp.float32)
            dv = plsc.bitcast(plsc.load_gather(tab_v, [di]), jnp.float32)
            dfa_v[b][pl.ds(16 * k, 16)] = dv - sv

        for tab_hbm, outs, group in (
                (pxy, (dxo, dyo), xy_group), (pzi, (dzo,), z_group)):
            pltpu.sync_copy(tab_hbm, tab_v)
            start_idx(0, 0)

            def pair_body(g2, _, outs=outs, group=group):
                for b in range(2):
                    g = 2 * g2 + b
                    nb = 1 - b

                    @pl.when(g + 1 < n_chunks)
                    def _():
                        start_idx(g + 1, nb)

                    wait_idx(b)

                    @pl.when(g >= 2)
                    def _():
                        wait_out(b, outs)

                    def edge_body(kk, _, b=b):
                        for u in range(_UNROLL):
                            group(b, _UNROLL * kk + u)
                        return _

                    lax.fori_loop(0, _B // (16 * _UNROLL), edge_body, None)
                    base = wid * e_per + g * _B
                    for o, dv in zip(outs, (dfa_v, dfb_v)):
                        pltpu.async_copy(dv[b], o.at[pl.ds(base, _B)],
                                         sem_o[b])
                return _

            lax.fori_loop(0, n_chunks // 2, pair_body, None)
            for b in range(2):
                wait_out(b, outs)

    return sc_call


def _tc_dist(dx, dy, dz, blk):
    n_edges = dx.shape[0]

    def body(x_ref, y_ref, z_ref, dist_ref):
        x, y, z = x_ref[...], y_ref[...], z_ref[...]
        dist_ref[...] = jnp.sqrt(x * x + y * y + z * z)

    return pl.pallas_call(
        body,
        grid=(n_edges // blk,),
        in_specs=[pl.BlockSpec((blk,), lambda i: (i,))] * 3,
        out_specs=pl.BlockSpec((blk,), lambda i: (i,)),
        out_shape=jax.ShapeDtypeStruct((n_edges,), jnp.float32),
    )(dx, dy, dz)


def _round_bf16_bits(f32_arr):
    u = lax.bitcast_convert_type(f32_arr, jnp.uint32)
    r = u + 0x7FFF + ((u >> 16) & 1)
    return r & jnp.uint32(0xFFFF0000)


def kernel(positions, edge_idx):
    n_nodes = positions.shape[0]
    n_edges = edge_idx.shape[0]
    xb = _round_bf16_bits(positions[:, 0])
    yb = _round_bf16_bits(positions[:, 1])
    pxy = lax.bitcast_convert_type(xb | (yb >> 16), jnp.int32)
    pzi = lax.bitcast_convert_type(positions[:, 2], jnp.int32)
    idx = edge_idx.astype(jnp.int32)
    sidx = idx[:, 0]
    didx = idx[:, 1]
    half = n_edges // 2
    sc = _make_sc_call(n_nodes, half)
    diffs, dists = [], []
    for lo in (0, half):
        dx, dy, dz = sc(pxy, pzi, sidx[lo:lo + half], didx[lo:lo + half])
        dists.append(_tc_dist(dx, dy, dz, 128000))
        diffs.append(jnp.stack([dx, dy, dz], axis=-1))
    return (jnp.concatenate(diffs, axis=0),
            jnp.concatenate(dists, axis=0))


# speedup vs baseline: 1.6722x; 1.6722x over previous
import functools

import jax
import jax.numpy as jnp
from jax import lax
from jax.experimental import pallas as pl
from jax.experimental.pallas import tpu as pltpu
from jax.experimental.pallas import tpu_sc as plsc

_NUM_CORES = 2
_NUM_SUBCORES = 16
_NW = _NUM_CORES * _NUM_SUBCORES
_B = 2000
_UNROLL = 5


def _make_sc_call(n_nodes, n_edges):
    e_per = n_edges // _NW
    n_chunks = e_per // _B
    mesh = plsc.VectorSubcoreMesh(
        core_axis_name="c", subcore_axis_name="s",
        num_cores=_NUM_CORES, num_subcores=_NUM_SUBCORES)

    @functools.partial(
        pl.kernel,
        out_type=tuple(
            jax.ShapeDtypeStruct((n_edges,), jnp.float32) for _ in range(3)),
        mesh=mesh,
        compiler_params=pltpu.CompilerParams(needs_layout_passes=False),
        scratch_types=[
            [pltpu.VMEM((_B,), jnp.int32)] * 2,
            [pltpu.VMEM((_B,), jnp.int32)] * 2,
            [pltpu.VMEM((_B,), jnp.float32)] * 2,
            [pltpu.VMEM((_B,), jnp.float32)] * 2,
            pltpu.VMEM((n_nodes,), jnp.int32),
            [pltpu.SemaphoreType.DMA] * 2,
            [pltpu.SemaphoreType.DMA] * 2,
        ],
    )
    def sc_call(pxy, pzi, sidx, didx, dxo, dyo, dzo,
                si_v, di_v, dfa_v, dfb_v, tab_v, sem_i, sem_o):
        wid = lax.axis_index("s") * _NUM_CORES + lax.axis_index("c")

        def start_idx(g, b):
            base = wid * e_per + g * _B
            pltpu.async_copy(sidx.at[pl.ds(base, _B)], si_v[b], sem_i[b])
            pltpu.async_copy(didx.at[pl.ds(base, _B)], di_v[b], sem_i[b])

        def wait_idx(b):
            pltpu.make_async_copy(sidx.at[pl.ds(0, _B)], si_v[b],
                                  sem_i[b]).wait()
            pltpu.make_async_copy(didx.at[pl.ds(0, _B)], di_v[b],
                                  sem_i[b]).wait()

        def wait_out(b, outs):
            for o, dv in zip(outs, (dfa_v, dfb_v)):
                pltpu.make_async_copy(dv[b], o.at[pl.ds(0, _B)],
                                      sem_o[b]).wait()

        def xy_group(b, k):
            si = si_v[b][pl.ds(16 * k, 16)]
            di = di_v[b][pl.ds(16 * k, 16)]
            sp = plsc.load_gather(tab_v, [si])
            dp = plsc.load_gather(tab_v, [di])
            hi = jnp.int32(-65536)
            sx = plsc.bitcast(sp & hi, jnp.float32)
            dxv = plsc.bitcast(dp & hi, jnp.float32)
            sy = plsc.bitcast(lax.shift_left(sp, 16), jnp.float32)
            dyv = plsc.bitcast(lax.shift_left(dp, 16), jnp.float32)
            dfa_v[b][pl.ds(16 * k, 16)] = dxv - sx
            dfb_v[b][pl.ds(16 * k, 16)] = dyv - sy

        def z_group(b, k):
            si = si_v[b][pl.ds(16 * k, 16)]
            di = di_v[b][pl.ds(16 * k, 16)]
            sv = plsc.bitcast(plsc.load_gather(tab_v, [si]), jnp.float32)
            dv = plsc.bitcast(plsc.load_gather(tab_v, [di]), jnp.float32)
            dfa_v[b][pl.ds(16 * k, 16)] = dv - sv

        for tab_hbm, outs, group in (
                (pxy, (dxo, dyo), xy_group), (pzi, (dzo,), z_group)):
            pltpu.sync_copy(tab_hbm, tab_v)
            start_idx(0, 0)

            def pair_body(g2, _, outs=outs, group=group):
                for b in range(2):
                    g = 2 * g2 + b
                    nb = 1 - b

                    @pl.when(g + 1 < n_chunks)
                    def _():
                        start_idx(g + 1, nb)

                    wait_idx(b)

                    @pl.when(g >= 2)
                    def _():
                        wait_out(b, outs)

                    @plsc.parallel_loop(0, _B // 16, unroll=_UNROLL)
                    def _(k, b=b):
                        group(b, k)
                    base = wid * e_per + g * _B
                    for o, dv in zip(outs, (dfa_v, dfb_v)):
                        pltpu.async_copy(dv[b], o.at[pl.ds(base, _B)],
                                         sem_o[b])
                return _

            lax.fori_loop(0, n_chunks // 2, pair_body, None)
            for b in range(2):
                wait_out(b, outs)

    return sc_call


def _tc_dist(dx, dy, dz, blk):
    n_edges = dx.shape[0]

    def body(x_ref, y_ref, z_ref, dist_ref):
        x, y, z = x_ref[...], y_ref[...], z_ref[...]
        dist_ref[...] = jnp.sqrt(x * x + y * y + z * z)

    return pl.pallas_call(
        body,
        grid=(n_edges // blk,),
        in_specs=[pl.BlockSpec((blk,), lambda i: (i,))] * 3,
        out_specs=pl.BlockSpec((blk,), lambda i: (i,)),
        out_shape=jax.ShapeDtypeStruct((n_edges,), jnp.float32),
    )(dx, dy, dz)


def _round_bf16_bits(f32_arr):
    u = lax.bitcast_convert_type(f32_arr, jnp.uint32)
    r = u + 0x7FFF + ((u >> 16) & 1)
    return r & jnp.uint32(0xFFFF0000)


def kernel(positions, edge_idx):
    n_nodes = positions.shape[0]
    n_edges = edge_idx.shape[0]
    xb = _round_bf16_bits(positions[:, 0])
    yb = _round_bf16_bits(positions[:, 1])
    pxy = lax.bitcast_convert_type(xb | (yb >> 16), jnp.int32)
    pzi = lax.bitcast_convert_type(positions[:, 2], jnp.int32)
    idx = edge_idx.astype(jnp.int32)
    sidx = idx[:, 0]
    didx = idx[:, 1]
    dx, dy, dz = _make_sc_call(n_nodes, n_edges)(pxy, pzi, sidx, didx)
    dist = _tc_dist(dx, dy, dz, 128000)
    diff = jnp.stack([dx, dy, dz], axis=-1)
    return diff, dist
